# Initial kernel scaffold; baseline (speedup 1.0000x reference)
import jax
import jax.numpy as jnp
from jax.experimental import pallas as pl


def kernel(x, embedding, W):
    raise NotImplementedError("write your pallas kernel here")


# speedup vs baseline: 1.3866x; 1.3866x over previous
import functools

import jax
import jax.numpy as jnp
from jax import lax
from jax.experimental import pallas as pl
from jax.experimental.pallas import tpu as pltpu
from jax.experimental.pallas import tpu_sc as plsc

N_ATOMS = 100000
HIDDEN = 128

NC = 2
NS = 16
NW = NC * NS

CHUNK = 128
NCHUNKS = (N_ATOMS + CHUNK - 1) // CHUNK
N_PAD_ROWS = 784
TAIL_ROWS = N_ATOMS - (NCHUNKS - 1) * CHUNK
MAX_CHUNKS_W = 25
_ROW_IDS = jnp.minimum(
    jnp.arange(NW)[:, None] * 24
    + jnp.minimum(jnp.arange(NW), 14)[:, None]
    + jnp.arange(MAX_CHUNKS_W)[None, :],
    N_PAD_ROWS - 1,
)


def _mm_body(emb_ref, w_ref, p_ref):
    p_ref[...] = lax.dot_general(
        emb_ref[...], w_ref[...],
        (((1,), (1,)), ((), ())),
        preferred_element_type=jnp.float32,
    )


def _fused_table(emb_pad, w_pad):
    return pl.pallas_call(
        _mm_body,
        out_shape=jax.ShapeDtypeStruct((HIDDEN, HIDDEN), jnp.float32),
    )(emb_pad, w_pad)


_sc_mesh = plsc.VectorSubcoreMesh(core_axis_name="c", subcore_axis_name="s")


@functools.partial(
    pl.kernel,
    out_type=jax.ShapeDtypeStruct((N_ATOMS, HIDDEN), jnp.float32),
    mesh=_sc_mesh,
    scratch_types=[
        pltpu.VMEM((MAX_CHUNKS_W, CHUNK), jnp.int32),
        pltpu.VMEM((CHUNK, HIDDEN), jnp.float32),
        pltpu.SemaphoreType.DMA,
    ],
)
def _sc_gather(table_hbm, idx_hbm, out_hbm, idx_v, rows_v, sem):
    w = lax.axis_index("s") * NC + lax.axis_index("c")
    start = 24 * w + jnp.minimum(w, 14)
    nfull = 24 + (w < 14).astype(jnp.int32) - (w == NW - 1).astype(jnp.int32)

    pltpu.sync_copy(idx_hbm.at[w], idx_v)

    def body(t, _):
        c = start + t
        pltpu.async_copy(table_hbm.at[idx_v.at[t]], rows_v, sem).wait()
        pltpu.sync_copy(rows_v, out_hbm.at[pl.ds(c * CHUNK, CHUNK)])
        return 0

    lax.fori_loop(0, nfull, body, 0)

    @pl.when(w == NW - 1)
    def _tail():
        t = MAX_CHUNKS_W - 2
        pltpu.async_copy(table_hbm.at[idx_v.at[t]], rows_v, sem).wait()
        pltpu.sync_copy(
            rows_v.at[pl.ds(0, TAIL_ROWS)],
            out_hbm.at[pl.ds((NCHUNKS - 1) * CHUNK, TAIL_ROWS)],
        )


def kernel(x, embedding, W):
    emb_pad = jnp.zeros((HIDDEN, HIDDEN), jnp.float32).at[:embedding.shape[0], :embedding.shape[1]].set(embedding)
    w_pad = jnp.zeros((HIDDEN, HIDDEN), jnp.float32).at[:, :W.shape[1]].set(W)
    table = _fused_table(emb_pad, w_pad)

    idx2d = jnp.zeros((N_PAD_ROWS * CHUNK,), jnp.int32).at[:N_ATOMS].set(x).reshape(N_PAD_ROWS, CHUNK)
    idx3d = jnp.take(idx2d, _ROW_IDS, axis=0)
    return _sc_gather(table, idx3d)


# speedup vs baseline: 1.3971x; 1.0076x over previous
import functools

import jax
import jax.numpy as jnp
from jax import lax
from jax.experimental import pallas as pl
from jax.experimental.pallas import tpu as pltpu
from jax.experimental.pallas import tpu_sc as plsc

N_ATOMS = 100000
HIDDEN = 128

NC = 2
NS = 16
NW = NC * NS

CHUNK = 128
NCHUNKS = (N_ATOMS + CHUNK - 1) // CHUNK
N_PAD_ROWS = 784
TAIL_ROWS = N_ATOMS - (NCHUNKS - 1) * CHUNK
MAX_CHUNKS_W = 25
_ROW_IDS = jnp.minimum(
    jnp.arange(NW)[:, None] * 24
    + jnp.minimum(jnp.arange(NW), 14)[:, None]
    + jnp.arange(MAX_CHUNKS_W)[None, :],
    N_PAD_ROWS - 1,
)


def _mm_body(emb_ref, w_ref, p_ref):
    p_ref[...] = lax.dot_general(
        emb_ref[...], w_ref[...],
        (((1,), (1,)), ((), ())),
        preferred_element_type=jnp.float32,
    )


def _fused_table(emb_pad, w_pad):
    return pl.pallas_call(
        _mm_body,
        out_shape=jax.ShapeDtypeStruct((HIDDEN, HIDDEN), jnp.float32),
    )(emb_pad, w_pad)


_sc_mesh = plsc.VectorSubcoreMesh(core_axis_name="c", subcore_axis_name="s")


@functools.partial(
    pl.kernel,
    out_type=jax.ShapeDtypeStruct((N_ATOMS, HIDDEN), jnp.float32),
    mesh=_sc_mesh,
    scratch_types=[
        pltpu.VMEM((MAX_CHUNKS_W, CHUNK), jnp.int32),
        pltpu.VMEM((2, CHUNK, HIDDEN), jnp.float32),
        pltpu.SemaphoreType.DMA,
        pltpu.SemaphoreType.DMA,
        pltpu.SemaphoreType.DMA,
    ],
)
def _sc_gather(table_hbm, idx_hbm, out_hbm, idx_v, rows_v, sem_g, sem_w0, sem_w1):
    w = lax.axis_index("s") * NC + lax.axis_index("c")
    start = 24 * w + jnp.minimum(w, 14)
    nfull = 24 + (w < 14).astype(jnp.int32) - (w == NW - 1).astype(jnp.int32)

    pltpu.sync_copy(idx_hbm.at[w], idx_v)

    def body(t, _):
        c = start + t
        b = lax.rem(t, 2)

        @pl.when(jnp.logical_and(t >= 2, b == 0))
        def _():
            pltpu.make_async_copy(table_hbm, rows_v.at[0], sem_w0).wait()

        @pl.when(jnp.logical_and(t >= 2, b == 1))
        def _():
            pltpu.make_async_copy(table_hbm, rows_v.at[1], sem_w1).wait()

        pltpu.async_copy(table_hbm.at[idx_v.at[t]], rows_v.at[b], sem_g).wait()

        @pl.when(b == 0)
        def _():
            pltpu.async_copy(rows_v.at[0], out_hbm.at[pl.ds(c * CHUNK, CHUNK)], sem_w0)

        @pl.when(b == 1)
        def _():
            pltpu.async_copy(rows_v.at[1], out_hbm.at[pl.ds(c * CHUNK, CHUNK)], sem_w1)

        return 0

    lax.fori_loop(0, nfull, body, 0)

    pltpu.make_async_copy(table_hbm, rows_v.at[0], sem_w0).wait()
    pltpu.make_async_copy(table_hbm, rows_v.at[1], sem_w1).wait()

    @pl.when(w == NW - 1)
    def _tail():
        t = MAX_CHUNKS_W - 2
        pltpu.async_copy(table_hbm.at[idx_v.at[t]], rows_v.at[0], sem_g).wait()
        pltpu.sync_copy(
            rows_v.at[0].at[pl.ds(0, TAIL_ROWS)],
            out_hbm.at[pl.ds((NCHUNKS - 1) * CHUNK, TAIL_ROWS)],
        )


def kernel(x, embedding, W):
    emb_pad = jnp.zeros((HIDDEN, HIDDEN), jnp.float32).at[:embedding.shape[0], :embedding.shape[1]].set(embedding)
    w_pad = jnp.zeros((HIDDEN, HIDDEN), jnp.float32).at[:, :W.shape[1]].set(W)
    table = _fused_table(emb_pad, w_pad)

    idx2d = jnp.zeros((N_PAD_ROWS * CHUNK,), jnp.int32).at[:N_ATOMS].set(x).reshape(N_PAD_ROWS, CHUNK)
    idx3d = jnp.take(idx2d, _ROW_IDS, axis=0)
    return _sc_gather(table, idx3d)


# speedup vs baseline: 1.4007x; 1.0026x over previous
import functools

import jax
import jax.numpy as jnp
from jax import lax
from jax.experimental import pallas as pl
from jax.experimental.pallas import tpu as pltpu
from jax.experimental.pallas import tpu_sc as plsc

N_ATOMS = 100000
HIDDEN = 128

NC = 2
NS = 16
NW = NC * NS

CHUNK = 128
NCHUNKS = (N_ATOMS + CHUNK - 1) // CHUNK
N_PAD_ROWS = 784
TAIL_ROWS = N_ATOMS - (NCHUNKS - 1) * CHUNK
MAX_CHUNKS_W = 25
_ROW_IDS = jnp.minimum(
    jnp.arange(NW)[:, None] * 24
    + jnp.minimum(jnp.arange(NW), 14)[:, None]
    + jnp.arange(MAX_CHUNKS_W)[None, :],
    N_PAD_ROWS - 1,
)


def _mm_body(emb_ref, w_ref, p_ref):
    p_ref[...] = lax.dot_general(
        emb_ref[...], w_ref[...],
        (((1,), (1,)), ((), ())),
        preferred_element_type=jnp.float32,
    )


def _fused_table(emb_pad, w_pad):
    return pl.pallas_call(
        _mm_body,
        out_shape=jax.ShapeDtypeStruct((HIDDEN, HIDDEN), jnp.float32),
    )(emb_pad, w_pad)


_sc_mesh = plsc.VectorSubcoreMesh(core_axis_name="c", subcore_axis_name="s")


@functools.partial(
    pl.kernel,
    out_type=jax.ShapeDtypeStruct((N_ATOMS, HIDDEN), jnp.float32),
    mesh=_sc_mesh,
    scratch_types=[
        pltpu.VMEM((MAX_CHUNKS_W, CHUNK), jnp.int32),
        pltpu.VMEM((4, CHUNK, HIDDEN), jnp.float32),
        pltpu.SemaphoreType.DMA((2,)),
        pltpu.SemaphoreType.DMA((4,)),
    ],
)
def _sc_gather(table_hbm, idx_hbm, out_hbm, idx_v, rows_v, sem_g, sem_w):
    w = lax.axis_index("s") * NC + lax.axis_index("c")
    start = 24 * w + jnp.minimum(w, 14)
    nfull = 24 + (w < 14).astype(jnp.int32) - (w == NW - 1).astype(jnp.int32)

    pltpu.sync_copy(idx_hbm.at[w], idx_v)

    pltpu.async_copy(table_hbm.at[idx_v.at[0]], rows_v.at[0], sem_g.at[0])

    def body(t, _):
        @pl.when(t + 1 < nfull)
        def _():
            b1 = lax.rem(t + 1, 4)

            @pl.when(t + 1 >= 4)
            def _():
                pltpu.make_async_copy(table_hbm, rows_v.at[b1], sem_w.at[b1]).wait()

            pltpu.async_copy(
                table_hbm.at[idx_v.at[t + 1]], rows_v.at[b1], sem_g.at[lax.rem(t + 1, 2)]
            )

        b = lax.rem(t, 4)
        pltpu.make_async_copy(
            table_hbm.at[idx_v.at[t]], rows_v.at[b], sem_g.at[lax.rem(t, 2)]
        ).wait()
        pltpu.async_copy(
            rows_v.at[b], out_hbm.at[pl.ds((start + t) * CHUNK, CHUNK)], sem_w.at[b]
        )
        return 0

    lax.fori_loop(0, nfull, body, 0)

    pltpu.make_async_copy(table_hbm, rows_v.at[0], sem_w.at[0]).wait()
    pltpu.make_async_copy(table_hbm, rows_v.at[1], sem_w.at[1]).wait()
    pltpu.make_async_copy(table_hbm, rows_v.at[2], sem_w.at[2]).wait()
    pltpu.make_async_copy(table_hbm, rows_v.at[3], sem_w.at[3]).wait()

    @pl.when(w == NW - 1)
    def _tail():
        t = MAX_CHUNKS_W - 2
        pltpu.async_copy(table_hbm.at[idx_v.at[t]], rows_v.at[0], sem_g.at[0]).wait()
        pltpu.sync_copy(
            rows_v.at[0].at[pl.ds(0, TAIL_ROWS)],
            out_hbm.at[pl.ds((NCHUNKS - 1) * CHUNK, TAIL_ROWS)],
        )


def kernel(x, embedding, W):
    emb_pad = jnp.zeros((HIDDEN, HIDDEN), jnp.float32).at[:embedding.shape[0], :embedding.shape[1]].set(embedding)
    w_pad = jnp.zeros((HIDDEN, HIDDEN), jnp.float32).at[:, :W.shape[1]].set(W)
    table = _fused_table(emb_pad, w_pad)

    idx2d = jnp.zeros((N_PAD_ROWS * CHUNK,), jnp.int32).at[:N_ATOMS].set(x).reshape(N_PAD_ROWS, CHUNK)
    idx3d = jnp.take(idx2d, _ROW_IDS, axis=0)
    return _sc_gather(table, idx3d)


# speedup vs baseline: 4.9187x; 3.5116x over previous
import functools

import jax
import jax.numpy as jnp
from jax import lax
from jax.experimental import pallas as pl
from jax.experimental.pallas import tpu as pltpu
from jax.experimental.pallas import tpu_sc as plsc

N_ATOMS = 100000
HIDDEN = 128

NC = 2
NS = 16
NW = NC * NS

CHUNK = 128
NCHUNKS = (N_ATOMS + CHUNK - 1) // CHUNK
N_PAD_ROWS = 784
TAIL_ROWS = N_ATOMS - (NCHUNKS - 1) * CHUNK
MAX_CHUNKS_W = 25
_ROW_IDS = jnp.minimum(
    jnp.arange(NW)[:, None] * 24
    + jnp.minimum(jnp.arange(NW), 14)[:, None]
    + jnp.arange(MAX_CHUNKS_W)[None, :],
    N_PAD_ROWS - 1,
)


def _mm_body(emb_ref, w_ref, p_ref):
    p_ref[...] = lax.dot_general(
        emb_ref[...], w_ref[...],
        (((1,), (1,)), ((), ())),
        preferred_element_type=jnp.float32,
    )


def _fused_table(emb_pad, w_pad):
    return pl.pallas_call(
        _mm_body,
        out_shape=jax.ShapeDtypeStruct((HIDDEN, HIDDEN), jnp.float32),
    )(emb_pad, w_pad)


_sc_mesh = plsc.VectorSubcoreMesh(core_axis_name="c", subcore_axis_name="s")


@functools.partial(
    pl.kernel,
    out_type=jax.ShapeDtypeStruct((N_ATOMS, HIDDEN), jnp.float32),
    mesh=_sc_mesh,
    scratch_types=[
        pltpu.VMEM((MAX_CHUNKS_W, CHUNK), jnp.int32),
        pltpu.VMEM((4, CHUNK, HIDDEN), jnp.float32),
        pltpu.VMEM_SHARED((HIDDEN, HIDDEN), jnp.float32),
        pltpu.SemaphoreType.DMA((2,)),
        pltpu.SemaphoreType.DMA((4,)),
    ],
)
def _sc_gather(table_hbm, idx_hbm, out_hbm, idx_v, rows_v, table_sh, sem_g, sem_w):
    w = lax.axis_index("s") * NC + lax.axis_index("c")
    start = 24 * w + jnp.minimum(w, 14)
    nfull = 24 + (w < 14).astype(jnp.int32) - (w == NW - 1).astype(jnp.int32)

    @pl.when(lax.axis_index("s") == 0)
    def _():
        pltpu.sync_copy(table_hbm, table_sh)

    pltpu.sync_copy(idx_hbm.at[w], idx_v)
    plsc.subcore_barrier()

    pltpu.async_copy(table_sh.at[idx_v.at[0]], rows_v.at[0], sem_g.at[0])

    def body(t, _):
        @pl.when(t + 1 < nfull)
        def _():
            b1 = lax.rem(t + 1, 4)

            @pl.when(t + 1 >= 4)
            def _():
                pltpu.make_async_copy(table_hbm, rows_v.at[b1], sem_w.at[b1]).wait()

            pltpu.async_copy(
                table_sh.at[idx_v.at[t + 1]], rows_v.at[b1], sem_g.at[lax.rem(t + 1, 2)]
            )

        b = lax.rem(t, 4)
        pltpu.make_async_copy(
            table_sh.at[idx_v.at[t]], rows_v.at[b], sem_g.at[lax.rem(t, 2)]
        ).wait()
        pltpu.async_copy(
            rows_v.at[b], out_hbm.at[pl.ds((start + t) * CHUNK, CHUNK)], sem_w.at[b]
        )
        return 0

    lax.fori_loop(0, nfull, body, 0)

    pltpu.make_async_copy(table_hbm, rows_v.at[0], sem_w.at[0]).wait()
    pltpu.make_async_copy(table_hbm, rows_v.at[1], sem_w.at[1]).wait()
    pltpu.make_async_copy(table_hbm, rows_v.at[2], sem_w.at[2]).wait()
    pltpu.make_async_copy(table_hbm, rows_v.at[3], sem_w.at[3]).wait()

    @pl.when(w == NW - 1)
    def _tail():
        t = MAX_CHUNKS_W - 2
        pltpu.async_copy(table_sh.at[idx_v.at[t]], rows_v.at[0], sem_g.at[0]).wait()
        pltpu.sync_copy(
            rows_v.at[0].at[pl.ds(0, TAIL_ROWS)],
            out_hbm.at[pl.ds((NCHUNKS - 1) * CHUNK, TAIL_ROWS)],
        )


def kernel(x, embedding, W):
    emb_pad = jnp.zeros((HIDDEN, HIDDEN), jnp.float32).at[:embedding.shape[0], :embedding.shape[1]].set(embedding)
    w_pad = jnp.zeros((HIDDEN, HIDDEN), jnp.float32).at[:, :W.shape[1]].set(W)
    table = _fused_table(emb_pad, w_pad)

    idx2d = jnp.zeros((N_PAD_ROWS * CHUNK,), jnp.int32).at[:N_ATOMS].set(x).reshape(N_PAD_ROWS, CHUNK)
    idx3d = jnp.take(idx2d, _ROW_IDS, axis=0)
    return _sc_gather(table, idx3d)


# speedup vs baseline: 5.5826x; 1.1350x over previous
import functools

import jax
import jax.numpy as jnp
from jax import lax
from jax.experimental import pallas as pl
from jax.experimental.pallas import tpu as pltpu
from jax.experimental.pallas import tpu_sc as plsc

N_ATOMS = 100000
N_ELEM = 100
HIDDEN = 128

NC = 2
NS = 16
NW = NC * NS

CHUNK = 128
NCHUNKS = (N_ATOMS + CHUNK - 1) // CHUNK
TAIL_ROWS = N_ATOMS - (NCHUNKS - 1) * CHUNK
MAX_CHUNKS_W = 25
IDX_STAGE = MAX_CHUNKS_W * CHUNK
IDX_STAGE_LAST = N_ATOMS - 758 * CHUNK


def _mm_body(emb_ref, w_ref, p_ref):
    p = lax.dot_general(
        emb_ref[...], w_ref[...],
        (((1,), (1,)), ((), ())),
        preferred_element_type=jnp.float32,
    )
    p_ref[...] = jnp.concatenate(
        [p, jnp.zeros((HIDDEN - N_ELEM, HIDDEN), jnp.float32)], axis=0
    )


def _fused_table(embedding, W):
    return pl.pallas_call(
        _mm_body,
        out_shape=jax.ShapeDtypeStruct((HIDDEN, HIDDEN), jnp.float32),
    )(embedding, W)


_sc_mesh = plsc.VectorSubcoreMesh(core_axis_name="c", subcore_axis_name="s")


@functools.partial(
    pl.kernel,
    out_type=jax.ShapeDtypeStruct((N_ATOMS, HIDDEN), jnp.float32),
    mesh=_sc_mesh,
    scratch_types=[
        pltpu.VMEM((IDX_STAGE,), jnp.int32),
        pltpu.VMEM((4, CHUNK, HIDDEN), jnp.float32),
        pltpu.VMEM_SHARED((HIDDEN, HIDDEN), jnp.float32),
        pltpu.SemaphoreType.DMA((2,)),
        pltpu.SemaphoreType.DMA((4,)),
    ],
)
def _sc_gather(table_hbm, x_hbm, out_hbm, idx_v, rows_v, table_sh, sem_g, sem_w):
    w = lax.axis_index("s") * NC + lax.axis_index("c")
    start = 24 * w + jnp.minimum(w, 14)
    nfull = 24 + (w < 14).astype(jnp.int32) - (w == NW - 1).astype(jnp.int32)

    @pl.when(lax.axis_index("s") == 0)
    def _():
        pltpu.sync_copy(table_hbm, table_sh)

    @pl.when(w < NW - 1)
    def _():
        pltpu.sync_copy(x_hbm.at[pl.ds(start * CHUNK, IDX_STAGE)], idx_v)

    @pl.when(w == NW - 1)
    def _():
        pltpu.sync_copy(
            x_hbm.at[pl.ds(start * CHUNK, IDX_STAGE_LAST)],
            idx_v.at[pl.ds(0, IDX_STAGE_LAST)],
        )

    plsc.subcore_barrier()

    pltpu.async_copy(
        table_sh.at[idx_v.at[pl.ds(0, CHUNK)]], rows_v.at[0], sem_g.at[0]
    )

    def body(t, _):
        @pl.when(t + 1 < nfull)
        def _():
            b1 = lax.rem(t + 1, 4)

            @pl.when(t + 1 >= 4)
            def _():
                pltpu.make_async_copy(table_hbm, rows_v.at[b1], sem_w.at[b1]).wait()

            pltpu.async_copy(
                table_sh.at[idx_v.at[pl.ds((t + 1) * CHUNK, CHUNK)]],
                rows_v.at[b1],
                sem_g.at[lax.rem(t + 1, 2)],
            )

        b = lax.rem(t, 4)
        pltpu.make_async_copy(
            table_sh.at[idx_v.at[pl.ds(t * CHUNK, CHUNK)]],
            rows_v.at[b],
            sem_g.at[lax.rem(t, 2)],
        ).wait()
        pltpu.async_copy(
            rows_v.at[b], out_hbm.at[pl.ds((start + t) * CHUNK, CHUNK)], sem_w.at[b]
        )
        return 0

    lax.fori_loop(0, nfull, body, 0)

    pltpu.make_async_copy(table_hbm, rows_v.at[0], sem_w.at[0]).wait()
    pltpu.make_async_copy(table_hbm, rows_v.at[1], sem_w.at[1]).wait()
    pltpu.make_async_copy(table_hbm, rows_v.at[2], sem_w.at[2]).wait()
    pltpu.make_async_copy(table_hbm, rows_v.at[3], sem_w.at[3]).wait()

    @pl.when(w == NW - 1)
    def _tail():
        off = (NCHUNKS - 1 - 758) * CHUNK
        pltpu.async_copy(
            table_sh.at[idx_v.at[pl.ds(off, TAIL_ROWS)]],
            rows_v.at[0].at[pl.ds(0, TAIL_ROWS)],
            sem_g.at[0],
        ).wait()
        pltpu.sync_copy(
            rows_v.at[0].at[pl.ds(0, TAIL_ROWS)],
            out_hbm.at[pl.ds((NCHUNKS - 1) * CHUNK, TAIL_ROWS)],
        )


def kernel(x, embedding, W):
    table = _fused_table(embedding, W)
    return _sc_gather(table, x)
